# baseline (speedup 1.0000x reference)
import functools

import jax
import jax.numpy as jnp
from jax import lax
from jax.experimental import pallas as pl
from jax.experimental.pallas import tpu as pltpu
from jax.experimental.pallas import tpu_sc as plsc

_B = 16384
_D = 32
_VP = 1_000_000
_VS = 100_000
_TAIL = (_VP // 128) * 128
_NC = 2
_NS = 16
_NW = _NC * _NS
_BPW = _B // _NW
_CH = 128
_NCH = _BPW // _CH
_RING = 8

_mesh = plsc.VectorSubcoreMesh(core_axis_name="c", subcore_axis_name="s")


@functools.partial(
    pl.kernel,
    out_type=jax.ShapeDtypeStruct((3 * _D, _B), jnp.float32),
    mesh=_mesh,
    scratch_types=[
        pltpu.VMEM((_BPW + 16,), jnp.int32),
        pltpu.VMEM((2, _NCH, _CH), jnp.int32),
        pltpu.VMEM((_NCH, _CH), jnp.int32),
        pltpu.VMEM((_CH, 128), jnp.float32),
        pltpu.VMEM((_RING * _D, 128), jnp.float32),
        pltpu.VMEM((3 * _D, _BPW), jnp.float32),
        pltpu.SemaphoreType.DMA,
        pltpu.SemaphoreType.DMA((_RING,)),
        pltpu.SemaphoreType.DMA,
    ],
    compiler_params=pltpu.CompilerParams(needs_layout_passes=False),
)
def _cat_embed(p_idx, b_idx, t_idx, wtp, wtail, ws, out,
               psv, sidx, tidx, srows, slot, cat, ssem, bsem, tsem):
    wid = lax.axis_index("s") * _NC + lax.axis_index("c")
    base = wid * _BPW
    iota = lax.iota(jnp.int32, 16)

    def splat(x):
        return jnp.full((16,), x, jnp.int32)

    def sread(i):
        return psv[pl.ds(i, 16)][0]

    for k in range(_NCH):
        pltpu.sync_copy(p_idx.at[pl.ds(base + k * _CH, _CH)],
                        psv.at[pl.ds(k * _CH, _CH)])
    for s in range(2):
        ih = b_idx if s == 0 else t_idx
        for k in range(_NCH):
            pltpu.sync_copy(ih.at[pl.ds(base + k * _CH, _CH)], sidx.at[s, k])

    for s in range(2):
        for k in range(_NCH):
            pltpu.async_copy(ws.at[sidx.at[s, k]], srows, ssem).wait()

            def small_grp(g, s=s, k=k):
                rowv = splat(g * 16) + iota
                colv = splat(k * _CH) + rowv
                for d in range(_D):
                    vals = plsc.load_gather(srows, [rowv, splat(s * _D + d)])
                    plsc.store_scatter(cat, [splat(_D + s * _D + d), colv],
                                       vals)

            pl.loop(0, _CH // 16)(small_grp)

    def fire(i):
        v = sread(i)
        cc = jnp.minimum(lax.shift_right_logical(v, 7), _TAIL // 128 - 1)
        c0 = pl.multiple_of(cc * 128, 128)
        slb = lax.rem(i, _RING) * _D
        pltpu.async_copy(
            wtp.at[:, pl.ds(c0, 128)],
            slot.at[pl.ds(slb, _D), :],
            bsem.at[lax.rem(i, _RING)],
        )

    def drain_descr(i):
        slb = lax.rem(i, _RING) * _D
        pltpu.make_async_copy(
            wtp.at[:, pl.ds(0, 128)],
            slot.at[pl.ds(slb, _D), :],
            bsem.at[lax.rem(i, _RING)],
        ).wait()

    for i in range(_RING):
        fire(i)

    def big_body(i):
        drain_descr(i)
        v = sread(i)
        cc = jnp.minimum(lax.shift_right_logical(v, 7), _TAIL // 128 - 1)
        c = jnp.minimum(v - cc * 128, 127)
        slb = lax.rem(i, _RING) * _D
        col = splat(c)
        for h in range(2):
            vals = plsc.load_gather(slot, [splat(slb + h * 16) + iota, col])
            plsc.store_scatter(cat, [splat(h * 16) + iota, splat(i)], vals)

        @pl.when(i + _RING < _BPW)
        def _():
            fire(i + _RING)

    pl.loop(0, _BPW)(big_body)

    for k in range(_NCH):
        for h in range(8):
            v16 = psv[pl.ds(k * _CH + h * 16, 16)]
            t16 = jnp.minimum(
                jnp.maximum(v16 - _TAIL, jnp.int32(0)), jnp.int32(63))
            tidx[k, pl.ds(h * 16, 16)] = t16
        pltpu.async_copy(wtail.at[tidx.at[k]], srows, tsem).wait()

        def tail_row(r, k=k):
            v = sread(k * _CH + r)

            @pl.when(v >= _TAIL)
            def _():
                col = splat(k * _CH + r)
                for h in range(2):
                    vals = srows[r, pl.ds(h * 16, 16)]
                    plsc.store_scatter(cat, [splat(h * 16) + iota, col], vals)

        pl.loop(0, _CH)(tail_row)

    pltpu.sync_copy(cat, out.at[:, pl.ds(base, _BPW)])


def kernel(positions, bet_sizing_id, topology, W_positions, W_bet_sizing_id, W_topology):
    wtp = W_positions.T
    wtail = jnp.pad(W_positions[_TAIL:], ((0, 0), (0, 96)))
    ws = jnp.pad(jnp.concatenate([W_bet_sizing_id, W_topology], axis=1),
                 ((0, 0), (0, 64)))
    out_t = _cat_embed(positions, bet_sizing_id, topology, wtp, wtail, ws)
    return out_t.T


# speedup vs baseline: 1.4082x; 1.4082x over previous
import functools

import jax
import jax.numpy as jnp
from jax import lax
from jax.experimental import pallas as pl
from jax.experimental.pallas import tpu as pltpu
from jax.experimental.pallas import tpu_sc as plsc

_B = 16384
_D = 32
_VP = 1_000_000
_NC = 2
_NS = 16
_NW = _NC * _NS
_BPW = _B // _NW
_CH = 128
_NCH = _BPW // _CH

_mesh = plsc.VectorSubcoreMesh(core_axis_name="c", subcore_axis_name="s")


@functools.partial(
    pl.kernel,
    out_type=jax.ShapeDtypeStruct((3 * _D, _B), jnp.float32),
    mesh=_mesh,
    scratch_types=[
        pltpu.VMEM((3, _NCH, _CH), jnp.int32),
        pltpu.VMEM((_NCH, _CH), jnp.int32),
        pltpu.VMEM((2, _CH, 128), jnp.float32),
        pltpu.VMEM((3 * _D, _BPW), jnp.float32),
        pltpu.SemaphoreType.DMA((2,)),
    ],
    compiler_params=pltpu.CompilerParams(needs_layout_passes=False),
)
def _cat_embed(p_idx, b_idx, t_idx, wp, ws, out, idx, brow, rows, cat, sem):
    wid = lax.axis_index("s") * _NC + lax.axis_index("c")
    base = wid * _BPW
    iota = lax.iota(jnp.int32, 16)

    def splat(x):
        return jnp.full((16,), x, jnp.int32)

    for f, ih in enumerate((p_idx, b_idx, t_idx)):
        for k in range(_NCH):
            pltpu.sync_copy(ih.at[pl.ds(base + k * _CH, _CH)], idx.at[f, k])
    for k in range(_NCH):
        for h in range(_CH // 16):
            v16 = idx[0, k, pl.ds(h * 16, 16)]
            brow[k, pl.ds(h * 16, 16)] = lax.shift_right_logical(v16, 2)

    tasks = [(0, k) for k in range(_NCH)] + \
            [(f, k) for f in (1, 2) for k in range(_NCH)]

    def start(t):
        f, k = tasks[t]
        buf = t % 2
        if f == 0:
            return pltpu.async_copy(wp.at[brow.at[k]], rows.at[buf], sem.at[buf])
        return pltpu.async_copy(ws.at[idx.at[f, k]], rows.at[buf], sem.at[buf])

    def extract(t):
        f, k = tasks[t]
        buf = t % 2

        if f == 0:
            def grp(g):
                rowv = splat(g * 16) + iota
                colv = splat(k * _CH) + rowv
                v16 = idx[0, k, pl.ds(g * 16, 16)]
                sub = jnp.bitwise_and(v16, jnp.int32(3)) * _D
                for d in range(_D):
                    vals = plsc.load_gather(rows.at[buf],
                                            [rowv, sub + splat(d)])
                    plsc.store_scatter(cat, [splat(d), colv], vals)
        else:
            def grp(g, f=f, k=k, buf=buf):
                rowv = splat(g * 16) + iota
                colv = splat(k * _CH) + rowv
                for d in range(_D):
                    vals = plsc.load_gather(
                        rows.at[buf], [rowv, splat((f - 1) * _D + d)])
                    plsc.store_scatter(cat, [splat(f * _D + d), colv], vals)

        pl.loop(0, _CH // 16)(grp)

    pend = [start(0), start(1)]
    for t in range(len(tasks)):
        pend[t % 2].wait()
        extract(t)
        if t + 2 < len(tasks):
            pend[t % 2] = start(t + 2)

    pltpu.sync_copy(cat, out.at[:, pl.ds(base, _BPW)])


def kernel(positions, bet_sizing_id, topology, W_positions, W_bet_sizing_id, W_topology):
    wp = W_positions.reshape(_VP // 4, 4 * _D)
    ws = jnp.pad(jnp.concatenate([W_bet_sizing_id, W_topology], axis=1),
                 ((0, 0), (0, 64)))
    out_t = _cat_embed(positions, bet_sizing_id, topology, wp, ws)
    return out_t.T


# speedup vs baseline: 2.4429x; 1.7348x over previous
import functools

import jax
import jax.numpy as jnp
from jax import lax
from jax.experimental import pallas as pl
from jax.experimental.pallas import tpu as pltpu
from jax.experimental.pallas import tpu_sc as plsc

_B = 16384
_D = 32
_VP = 1_000_000
_NC = 2
_NS = 16
_NW = _NC * _NS
_BPW = _B // _NW
_CH = 128
_NCH = _BPW // _CH

_mesh = plsc.VectorSubcoreMesh(core_axis_name="c", subcore_axis_name="s")


@functools.partial(
    pl.kernel,
    out_type=jax.ShapeDtypeStruct((3 * _D, _B), jnp.float32),
    mesh=_mesh,
    scratch_types=[
        pltpu.VMEM((3, _NCH, _CH), jnp.int32),
        pltpu.VMEM((_NCH, _CH), jnp.int32),
        pltpu.VMEM((2, _CH, 128), jnp.float32),
        pltpu.VMEM((3 * _D, _BPW), jnp.float32),
        pltpu.SemaphoreType.DMA((2,)),
    ],
    compiler_params=pltpu.CompilerParams(needs_layout_passes=False),
)
def _cat_embed(p_idx, b_idx, t_idx, wp, ws, out, idx, brow, rows, cat, sem):
    wid = lax.axis_index("s") * _NC + lax.axis_index("c")
    base = wid * _BPW
    iota = lax.iota(jnp.int32, 16)

    def splat(x):
        return jnp.full((16,), x, jnp.int32)

    for f, ih in enumerate((p_idx, b_idx, t_idx)):
        for k in range(_NCH):
            pltpu.sync_copy(ih.at[pl.ds(base + k * _CH, _CH)], idx.at[f, k])
    for k in range(_NCH):
        for h in range(_CH // 16):
            v16 = idx[0, k, pl.ds(h * 16, 16)]
            brow[k, pl.ds(h * 16, 16)] = (
                lax.shift_left(lax.shift_right_logical(v16, 14), 12)
                + jnp.bitwise_and(v16, jnp.int32(4095)))

    tasks = [(0, k) for k in range(_NCH)] + \
            [(f, k) for f in (1, 2) for k in range(_NCH)]

    def start(t):
        f, k = tasks[t]
        buf = t % 2
        if f == 0:
            return pltpu.async_copy(wp.at[brow.at[k]], rows.at[buf], sem.at[buf])
        return pltpu.async_copy(ws.at[idx.at[f, k]], rows.at[buf], sem.at[buf])

    def extract(t):
        f, k = tasks[t]
        buf = t % 2

        if f == 0:
            def grp(g):
                rowv = splat(g * 16) + iota
                colv = splat(k * _CH) + rowv
                v16 = idx[0, k, pl.ds(g * 16, 16)]
                sub = jnp.bitwise_and(
                    lax.shift_right_logical(v16, 12), jnp.int32(3)) * _D
                for d in range(_D):
                    vals = plsc.load_gather(rows.at[buf],
                                            [rowv, sub + splat(d)])
                    plsc.store_scatter(cat, [splat(d), colv], vals)
        else:
            def grp(g, f=f, k=k, buf=buf):
                rowv = splat(g * 16) + iota
                colv = splat(k * _CH) + rowv
                for d in range(_D):
                    vals = plsc.load_gather(
                        rows.at[buf], [rowv, splat((f - 1) * _D + d)])
                    plsc.store_scatter(cat, [splat(f * _D + d), colv], vals)

        pl.loop(0, _CH // 16)(grp)

    pend = [start(0), start(1)]
    for t in range(len(tasks)):
        pend[t % 2].wait()
        extract(t)
        if t + 2 < len(tasks):
            pend[t % 2] = start(t + 2)

    pltpu.sync_copy(cat, out.at[:, pl.ds(base, _BPW)])


_LB = 16384
_NBLK = (_VP + _LB - 1) // _LB


def _relayout_body(wt_ref, out_ref):
    xt = wt_ref[...].T
    q = _LB // 4
    out_ref[...] = jnp.concatenate(
        [xt[j * q:(j + 1) * q] for j in range(4)], axis=1)


_relayout = pl.pallas_call(
    _relayout_body,
    grid=(_NBLK,),
    in_specs=[pl.BlockSpec((_D, _LB), lambda i: (0, i))],
    out_specs=pl.BlockSpec((_LB // 4, 4 * _D), lambda i: (i, 0)),
    out_shape=jax.ShapeDtypeStruct((_NBLK * _LB // 4, 4 * _D), jnp.float32),
)


def kernel(positions, bet_sizing_id, topology, W_positions, W_bet_sizing_id, W_topology):
    wp = _relayout(W_positions.T)
    ws = jnp.pad(jnp.concatenate([W_bet_sizing_id, W_topology], axis=1),
                 ((0, 0), (0, 64)))
    out_t = _cat_embed(positions, bet_sizing_id, topology, wp, ws)
    return out_t.T


# speedup vs baseline: 2.4513x; 1.0034x over previous
import functools

import jax
import jax.numpy as jnp
from jax import lax
from jax.experimental import pallas as pl
from jax.experimental.pallas import tpu as pltpu
from jax.experimental.pallas import tpu_sc as plsc

_B = 16384
_D = 32
_VP = 1_000_000
_NC = 2
_NS = 16
_NW = _NC * _NS
_BPW = _B // _NW
_CH = 128
_NCH = _BPW // _CH

_mesh = plsc.VectorSubcoreMesh(core_axis_name="c", subcore_axis_name="s")


@functools.partial(
    pl.kernel,
    out_type=jax.ShapeDtypeStruct((3 * _D, _B), jnp.float32),
    mesh=_mesh,
    scratch_types=[
        pltpu.VMEM((3, _NCH, _CH), jnp.int32),
        pltpu.VMEM((_NCH, _CH), jnp.int32),
        pltpu.VMEM((2, _CH, 128), jnp.float32),
        pltpu.VMEM((3 * _D, _BPW), jnp.float32),
        pltpu.SemaphoreType.DMA((2,)),
    ],
    compiler_params=pltpu.CompilerParams(needs_layout_passes=False),
)
def _cat_embed(p_idx, b_idx, t_idx, wp, ws, out, idx, brow, rows, cat, sem):
    wid = lax.axis_index("s") * _NC + lax.axis_index("c")
    base = wid * _BPW
    iota = lax.iota(jnp.int32, 16)

    def splat(x):
        return jnp.full((16,), x, jnp.int32)

    for f, ih in enumerate((p_idx, b_idx, t_idx)):
        for k in range(_NCH):
            pltpu.sync_copy(ih.at[pl.ds(base + k * _CH, _CH)], idx.at[f, k])
    for k in range(_NCH):
        for h in range(_CH // 16):
            v16 = idx[0, k, pl.ds(h * 16, 16)]
            brow[k, pl.ds(h * 16, 16)] = (
                lax.shift_left(lax.shift_right_logical(v16, 15), 13)
                + jnp.bitwise_and(v16, jnp.int32(8191)))

    tasks = [(0, k) for k in range(_NCH)] + \
            [(f, k) for f in (1, 2) for k in range(_NCH)]

    def start(t):
        f, k = tasks[t]
        buf = t % 2
        if f == 0:
            return pltpu.async_copy(wp.at[brow.at[k]], rows.at[buf], sem.at[buf])
        return pltpu.async_copy(ws.at[idx.at[f, k]], rows.at[buf], sem.at[buf])

    def extract(t):
        f, k = tasks[t]
        buf = t % 2

        if f == 0:
            def grp(g):
                rowv = splat(g * 16) + iota
                colv = splat(k * _CH) + rowv
                v16 = idx[0, k, pl.ds(g * 16, 16)]
                sub = jnp.bitwise_and(
                    lax.shift_right_logical(v16, 13), jnp.int32(3)) * _D
                for d in range(_D):
                    vals = plsc.load_gather(rows.at[buf],
                                            [rowv, sub + splat(d)])
                    plsc.store_scatter(cat, [splat(d), colv], vals)
        else:
            def grp(g, f=f, k=k, buf=buf):
                rowv = splat(g * 16) + iota
                colv = splat(k * _CH) + rowv
                for d in range(_D):
                    vals = plsc.load_gather(
                        rows.at[buf], [rowv, splat((f - 1) * _D + d)])
                    plsc.store_scatter(cat, [splat(f * _D + d), colv], vals)

        pl.loop(0, _CH // 16)(grp)

    pend = [start(0), start(1)]
    for t in range(len(tasks)):
        pend[t % 2].wait()
        extract(t)
        if t + 2 < len(tasks):
            pend[t % 2] = start(t + 2)

    pltpu.sync_copy(cat, out.at[:, pl.ds(base, _BPW)])


_LB = 32768
_NBLK = (_VP + _LB - 1) // _LB


def _relayout_body(wt_ref, out_ref):
    xt = wt_ref[...].T
    q = _LB // 4
    out_ref[...] = jnp.concatenate(
        [xt[j * q:(j + 1) * q] for j in range(4)], axis=1)


_relayout = pl.pallas_call(
    _relayout_body,
    grid=(_NBLK,),
    in_specs=[pl.BlockSpec((_D, _LB), lambda i: (0, i))],
    out_specs=pl.BlockSpec((_LB // 4, 4 * _D), lambda i: (i, 0)),
    out_shape=jax.ShapeDtypeStruct((_NBLK * _LB // 4, 4 * _D), jnp.float32),
)


def kernel(positions, bet_sizing_id, topology, W_positions, W_bet_sizing_id, W_topology):
    wp = _relayout(W_positions.T)
    ws = jnp.pad(jnp.concatenate([W_bet_sizing_id, W_topology], axis=1),
                 ((0, 0), (0, 64)))
    out_t = _cat_embed(positions, bet_sizing_id, topology, wp, ws)
    return out_t.T
